# baseline (speedup 1.0000x reference)
import functools

import jax
import jax.numpy as jnp
from jax import lax
from jax.experimental import pallas as pl
from jax.experimental.pallas import tpu as pltpu
from jax.experimental.pallas import tpu_sc as plsc

N = 10000
E = 320000
D = 128
NS = 100
B = 100

NUM_CORES = 2
NUM_SUBCORES = 16
NUM_WORKERS = NUM_CORES * NUM_SUBCORES

CHUNK = 128
CPT = 80
E_PAD = NUM_WORKERS * CPT * CHUNK
PAD = E_PAD - E
TRASH = 8
TOT_CHUNKS = E_PAD // CHUNK
IDX_STAGES = ((0, 32), (32, 32), (64, 16))
IDX_BLK = 32

ZBLK = 200
NZB = N // ZBLK
ZB_ITERS = -(-NZB // NUM_SUBCORES)


def _sc_agg_body(x_hbm, z_hbm, src_hbm, dst_hbm, out_hbm,
                 srcblk, dstblk, rows, acc, sem0, sem1):
    cid = lax.axis_index("c")
    sid = lax.axis_index("s")
    wid = cid * NUM_SUBCORES + sid

    def stage_idx(k, stage, cnt):
        pltpu.sync_copy(src_hbm.at[pl.ds(wid * CPT + stage, cnt)],
                        srcblk.at[k % 2, pl.ds(0, cnt)])
        pltpu.sync_copy(dst_hbm.at[pl.ds(wid * CPT + stage, cnt)],
                        dstblk.at[k % 2, pl.ds(0, cnt)])

    stage_idx(0, *IDX_STAGES[0])

    def zblock(k, _):
        blk = k * NUM_SUBCORES + sid

        @pl.when(blk < NZB)
        def _():
            pltpu.sync_copy(z_hbm.at[pl.ds(blk * ZBLK, ZBLK)],
                            acc.at[pl.ds(blk * ZBLK, ZBLK)])
        return 0
    lax.fori_loop(0, ZB_ITERS, zblock, 0)
    plsc.subcore_barrier()

    rows0 = rows.at[0]
    rows1 = rows.at[1]
    for k, (stage, cnt) in enumerate(IDX_STAGES):
        sb = srcblk.at[k % 2]
        db = dstblk.at[k % 2]
        pltpu.async_copy(x_hbm.at[sb.at[0]], rows0, sem0)
        if k + 1 < len(IDX_STAGES):
            stage_idx(k + 1, *IDX_STAGES[k + 1])

        def pair_body(j, _):
            c0 = 2 * j
            c1 = 2 * j + 1
            pltpu.async_copy(x_hbm.at[sb.at[c1]], rows1, sem1)
            pltpu.make_async_copy(x_hbm.at[sb.at[c0]], rows0, sem0).wait()
            pltpu.sync_copy(rows0, acc.at[db.at[c0]], add=True)

            @pl.when(c1 + 1 < cnt)
            def _():
                pltpu.async_copy(
                    x_hbm.at[sb.at[jnp.minimum(c1 + 1, cnt - 1)]], rows0, sem0)
            pltpu.make_async_copy(x_hbm.at[sb.at[c1]], rows1, sem1).wait()
            pltpu.sync_copy(rows1, acc.at[db.at[c1]], add=True)
            return 0
        lax.fori_loop(0, cnt // 2, pair_body, 0)

    plsc.subcore_barrier()

    def wblock(k, _):
        blk = k * NUM_SUBCORES + sid

        @pl.when(blk < NZB)
        def _():
            pltpu.sync_copy(acc.at[pl.ds(blk * ZBLK, ZBLK)],
                            out_hbm.at[cid, pl.ds(blk * ZBLK, ZBLK)])
        return 0
    lax.fori_loop(0, ZB_ITERS, wblock, 0)


@functools.cache
def _sc_agg():
    return pl.kernel(
        _sc_agg_body,
        mesh=plsc.VectorSubcoreMesh(
            core_axis_name="c", subcore_axis_name="s",
            num_cores=NUM_CORES, num_subcores=NUM_SUBCORES),
        out_type=jax.ShapeDtypeStruct((NUM_CORES, N, D), jnp.float32),
        scratch_types=[
            pltpu.VMEM((2, IDX_BLK, CHUNK), jnp.int32),
            pltpu.VMEM((2, IDX_BLK, CHUNK), jnp.int32),
            pltpu.VMEM((2, CHUNK, D), jnp.float32),
            pltpu.VMEM_SHARED((N + TRASH, D), jnp.float32),
            pltpu.SemaphoreType.DMA,
            pltpu.SemaphoreType.DMA,
        ],
    )


ROWS_A = 1000


def _tc_a_body(p_ref, wg_ref, wo_ref, uw_ref, od_ref, u_ref):
    agg = p_ref[0] + p_ref[1]
    h = jnp.maximum(jnp.dot(agg, wg_ref[...],
                            preferred_element_type=jnp.float32), 0.0)
    od_ref[...] = jnp.dot(h, wo_ref[...], preferred_element_type=jnp.float32)
    u = (jnp.sum(h, axis=1) * (uw_ref[0] / D)
         + jnp.sum(agg, axis=1) * (uw_ref[1] / D))
    u_ref[...] = u.reshape(ROWS_A, 1)


def _tc_a(partials, W_gnn, W_od, utility_w):
    grid = N // ROWS_A
    return pl.pallas_call(
        _tc_a_body,
        grid=(grid,),
        in_specs=[
            pl.BlockSpec((NUM_CORES, ROWS_A, D), lambda i: (0, i, 0)),
            pl.BlockSpec((D, D), lambda i: (0, 0)),
            pl.BlockSpec((D, NS), lambda i: (0, 0)),
            pl.BlockSpec(memory_space=pltpu.SMEM),
        ],
        out_specs=[
            pl.BlockSpec((ROWS_A, NS), lambda i: (i, 0)),
            pl.BlockSpec((ROWS_A, 1), lambda i: (i, 0)),
        ],
        out_shape=[
            jax.ShapeDtypeStruct((N, NS), jnp.float32),
            jax.ShapeDtypeStruct((N, 1), jnp.float32),
        ],
    )(partials, W_gnn, W_od, utility_w)


def _tc_b_body(od_ref, we_ref, be_ref, wd_ref, bd_ref, u_ref, out_ref):
    od = od_ref[...]
    lat = jnp.maximum(jnp.dot(od, we_ref[...],
                              preferred_element_type=jnp.float32)
                      + be_ref[...], 0.0)
    rec = jnp.dot(lat, wd_ref[...],
                  preferred_element_type=jnp.float32) + bd_ref[...]
    u = u_ref[...]
    m = jnp.max(u, axis=1, keepdims=True)
    e = jnp.exp(u - m)
    p = e / jnp.sum(e, axis=1, keepdims=True)
    out_ref[:, :NS] = p
    out_ref[:, NS:] = rec


def _tc_b(od_flat, W_enc, b_enc, W_dec, b_dec, u):
    lat = W_enc.shape[1]
    return pl.pallas_call(
        _tc_b_body,
        out_shape=jax.ShapeDtypeStruct((B, NS + N), jnp.float32),
    )(od_flat, W_enc, b_enc.reshape(1, lat), W_dec, b_dec.reshape(1, N), u)


def kernel(x_od, edge_index, W_gnn, W_od, W_enc, b_enc, W_dec, b_dec, utility_w):
    pad_src = jnp.arange(PAD, dtype=jnp.int32) % N
    pad_dst = N + (jnp.arange(PAD, dtype=jnp.int32) % TRASH)
    src = jnp.concatenate([edge_index[0], pad_src]).reshape(TOT_CHUNKS, CHUNK)
    dst = jnp.concatenate([edge_index[1], pad_dst]).reshape(TOT_CHUNKS, CHUNK)
    zeros = jnp.zeros((N, D), jnp.float32)
    partials = _sc_agg()(x_od, zeros, src, dst)
    od, u = _tc_a(partials, W_gnn, W_od, utility_w)
    od_flat = od.reshape(B, NS * NS)
    u2 = u.reshape(B, NS)
    return _tc_b(od_flat, W_enc, b_enc, W_dec, b_dec, u2)
